# baseline (speedup 1.0000x reference)
import functools

import jax
import jax.numpy as jnp
from jax import lax
from jax.experimental import pallas as pl
from jax.experimental.pallas import tpu as pltpu
from jax.experimental.pallas import tpu_sc as plsc

NZ = 256
NX = 256
NY = 256
NN = NX * NY
NWORK = 32
NPW = NN // NWORK
CH = 128
NCHUNK = NPW // CH
NG = CH // 16
SIG0 = 0.8
HALF_EPOCHS = 50.0

_mesh = plsc.VectorSubcoreMesh(core_axis_name="c", subcore_axis_name="s")


@functools.partial(
    pl.kernel,
    mesh=_mesh,
    compiler_params=pltpu.CompilerParams(needs_layout_passes=False),
    out_type=(
        jax.ShapeDtypeStruct((NWORK, 16), jnp.float32),
        jax.ShapeDtypeStruct((NWORK, 16), jnp.int32),
    ),
    scratch_types=[
        pltpu.VMEM((NZ,), jnp.float32),
        pltpu.VMEM((CH, NZ), jnp.float32),
        pltpu.VMEM((CH * 16,), jnp.float32),
        pltpu.VMEM((16,), jnp.float32),
        pltpu.VMEM((16,), jnp.int32),
    ],
)
def _sc_scan(x_hbm, w_hbm, val_hbm, idx_hbm, xb, wb, accb, vout, iout):
    cid = lax.axis_index("c")
    sid = lax.axis_index("s")
    wid = sid * 2 + cid
    base = wid * NPW

    pltpu.sync_copy(x_hbm, xb)

    ivec = lax.iota(jnp.int32, 16)
    ivec16 = ivec * 16
    zero16 = jnp.zeros((16,), jnp.float32)

    def zbody(zc, _):
        zoff = zc * 16
        xv = xb[pl.ds(zoff, 16)]
        for n in range(CH):
            wv = wb[n, pl.ds(zoff, 16)]
            d = wv - xv
            plsc.addupdate(accb.at[pl.ds(n * 16, 16)], d * d)
        return 0

    def cbody(ci, carry):
        best, bidx = carry
        row0 = base + ci * CH
        pltpu.sync_copy(w_hbm.at[pl.ds(row0, CH)], wb)
        for n in range(CH):
            accb[pl.ds(n * 16, 16)] = zero16
        lax.fori_loop(0, NZ // 16, zbody, 0)
        for g in range(NG):
            tot = None
            for c in range(16):
                col = plsc.load_gather(accb, [ivec16 + (g * 256 + c)])
                tot = col if tot is None else tot + col
            gid = ivec + (row0 + g * 16)
            m = tot < best
            best = jnp.where(m, tot, best)
            bidx = jnp.where(m, gid, bidx)
        return best, bidx

    best0 = jnp.full((16,), jnp.inf, jnp.float32)
    bidx0 = jnp.zeros((16,), jnp.int32)
    best, bidx = lax.fori_loop(0, NCHUNK, cbody, (best0, bidx0))

    vout[...] = best
    iout[...] = bidx
    pltpu.sync_copy(vout, val_hbm.at[wid])
    pltpu.sync_copy(iout, idx_hbm.at[wid])


def _tc_finish_body(v_ref, i_ref, c_ref, o_ref):
    v = v_ref[...]
    ii = i_ref[...]
    minv = jnp.min(v)
    cand = jnp.where(v == minv, ii, jnp.int32(NN))
    flat = jnp.min(cand)
    pf = (flat // NY).astype(jnp.float32)
    qf = (flat % NY).astype(jnp.float32)
    rows = lax.broadcasted_iota(jnp.int32, (NX, NY), 0).astype(jnp.float32)
    cols = lax.broadcasted_iota(jnp.int32, (NX, NY), 1).astype(jnp.float32)
    nv = c_ref[0]
    o_ref[...] = jnp.exp(((rows - pf) ** 2 + (cols - qf) ** 2) * nv)


_tc_finish = pl.pallas_call(
    _tc_finish_body,
    out_shape=jax.ShapeDtypeStruct((NX, NY), jnp.float32),
    in_specs=[
        pl.BlockSpec(memory_space=pltpu.VMEM),
        pl.BlockSpec(memory_space=pltpu.VMEM),
        pl.BlockSpec(memory_space=pltpu.SMEM),
    ],
    out_specs=pl.BlockSpec(memory_space=pltpu.VMEM),
)


def kernel(x, W, t):
    w2 = W.reshape(NN, NZ)
    tf = jnp.asarray(t).astype(jnp.float32)
    dec = SIG0 / (1.0 + tf / HALF_EPOCHS)
    ninv = (-1.0 / (2.0 * dec * dec)).reshape(1).astype(jnp.float32)
    vals, idxs = _sc_scan(x, w2)
    return _tc_finish(vals, idxs, ninv)


# speedup vs baseline: 1.9617x; 1.9617x over previous
import functools

import jax
import jax.numpy as jnp
from jax import lax
from jax.experimental import pallas as pl
from jax.experimental.pallas import tpu as pltpu
from jax.experimental.pallas import tpu_sc as plsc

NZ = 256
NX = 256
NY = 256
NN = NX * NY
NWORK = 32
NPW = NN // NWORK
CH = 128
NCHUNK = NPW // CH
NLANE = 16
NU = 8
NGRP = CH // NU
SIG0 = 0.8
HALF_EPOCHS = 50.0

_mesh = plsc.VectorSubcoreMesh(core_axis_name="c", subcore_axis_name="s")


@functools.partial(
    pl.kernel,
    mesh=_mesh,
    compiler_params=pltpu.CompilerParams(needs_layout_passes=False),
    out_type=(
        jax.ShapeDtypeStruct((NWORK, NLANE), jnp.float32),
        jax.ShapeDtypeStruct((NWORK, NLANE), jnp.int32),
    ),
    scratch_types=[
        pltpu.VMEM((NZ,), jnp.float32),
        pltpu.VMEM((2, CH * NZ), jnp.float32),
        pltpu.VMEM((CH * NLANE,), jnp.float32),
        pltpu.VMEM((NLANE,), jnp.float32),
        pltpu.VMEM((NLANE,), jnp.int32),
        pltpu.SemaphoreType.DMA,
        pltpu.SemaphoreType.DMA,
    ],
)
def _sc_scan(x_hbm, w_hbm, val_hbm, idx_hbm, xb, wb, accb, vout, iout,
             sem0, sem1):
    cid = lax.axis_index("c")
    sid = lax.axis_index("s")
    wid = sid * 2 + cid
    base = wid * NPW
    sems = (sem0, sem1)

    pltpu.sync_copy(x_hbm, xb)
    xs = [xb[pl.ds(zc * NLANE, NLANE)] for zc in range(NZ // NLANE)]

    ivec = lax.iota(jnp.int32, NLANE)
    ivec16 = ivec * NLANE

    def chunk_src(ci):
        return w_hbm.at[pl.ds((base + ci * CH) * NZ, CH * NZ)]

    pltpu.async_copy(chunk_src(0), wb.at[0], sem0)

    def cbody(kk, carry):
        best, bidx = carry
        for b in (0, 1):
            ci = kk * 2 + b
            pltpu.make_async_copy(chunk_src(ci), wb.at[b], sems[b]).wait()

            @pl.when(ci + 1 < NCHUNK)
            def _():
                pltpu.async_copy(chunk_src(ci + 1), wb.at[1 - b], sems[1 - b])

            def gbody(g, _, b=b):
                goff = g * (NU * NZ)
                accs = []
                for k in range(NU):
                    acc = None
                    noff = goff + k * NZ
                    for zc in range(NZ // NLANE):
                        wv = wb[b, pl.ds(noff + zc * NLANE, NLANE)]
                        d = wv - xs[zc]
                        m = d * d
                        acc = m if acc is None else acc + m
                    accs.append(acc)
                aoff = g * (NU * NLANE)
                for k in range(NU):
                    accb[pl.ds(aoff + k * NLANE, NLANE)] = accs[k]
                return 0

            lax.fori_loop(0, NGRP, gbody, 0)

            row0 = base + ci * CH
            for g in range(CH // NLANE):
                tot = None
                for c in range(NLANE):
                    col = plsc.load_gather(
                        accb, [ivec16 + (g * NLANE * NLANE + c)])
                    tot = col if tot is None else tot + col
                gid = ivec + (row0 + g * NLANE)
                m = tot < best
                best = jnp.where(m, tot, best)
                bidx = jnp.where(m, gid, bidx)
        return best, bidx

    best = jnp.full((NLANE,), jnp.inf, jnp.float32)
    bidx = jnp.zeros((NLANE,), jnp.int32)
    best, bidx = lax.fori_loop(0, NCHUNK // 2, cbody, (best, bidx))

    vout[...] = best
    iout[...] = bidx
    pltpu.sync_copy(vout, val_hbm.at[wid])
    pltpu.sync_copy(iout, idx_hbm.at[wid])


def _tc_finish_body(v_ref, i_ref, c_ref, o_ref):
    v = v_ref[...]
    ii = i_ref[...]
    minv = jnp.min(v)
    cand = jnp.where(v == minv, ii, jnp.int32(NN))
    flat = jnp.min(cand)
    pf = (flat // NY).astype(jnp.float32)
    qf = (flat % NY).astype(jnp.float32)
    rows = lax.broadcasted_iota(jnp.int32, (NX, NY), 0).astype(jnp.float32)
    cols = lax.broadcasted_iota(jnp.int32, (NX, NY), 1).astype(jnp.float32)
    nv = c_ref[0]
    o_ref[...] = jnp.exp(((rows - pf) ** 2 + (cols - qf) ** 2) * nv)


_tc_finish = pl.pallas_call(
    _tc_finish_body,
    out_shape=jax.ShapeDtypeStruct((NX, NY), jnp.float32),
    in_specs=[
        pl.BlockSpec(memory_space=pltpu.VMEM),
        pl.BlockSpec(memory_space=pltpu.VMEM),
        pl.BlockSpec(memory_space=pltpu.SMEM),
    ],
    out_specs=pl.BlockSpec(memory_space=pltpu.VMEM),
)


def kernel(x, W, t):
    wf = W.reshape(NN * NZ)
    tf = jnp.asarray(t).astype(jnp.float32)
    dec = SIG0 / (1.0 + tf / HALF_EPOCHS)
    ninv = (-1.0 / (2.0 * dec * dec)).reshape(1).astype(jnp.float32)
    vals, idxs = _sc_scan(x, wf)
    return _tc_finish(vals, idxs, ninv)


# speedup vs baseline: 2.3124x; 1.1788x over previous
import functools

import jax
import jax.numpy as jnp
from jax import lax
from jax.experimental import pallas as pl
from jax.experimental.pallas import tpu as pltpu
from jax.experimental.pallas import tpu_sc as plsc

NZ = 256
NX = 256
NY = 256
NN = NX * NY
NWORK = 32
S_SC = 16384
NPW = S_SC // NWORK
CH = 128
NCHUNK = NPW // CH
NLANE = 16
NU = 8
NGRP = CH // NU
BTC = 2048
GTC = (NN - S_SC) // BTC
SIG0 = 0.8
HALF_EPOCHS = 50.0

_mesh = plsc.VectorSubcoreMesh(core_axis_name="c", subcore_axis_name="s")


@functools.partial(
    pl.kernel,
    mesh=_mesh,
    compiler_params=pltpu.CompilerParams(needs_layout_passes=False),
    out_type=(
        jax.ShapeDtypeStruct((NWORK, NLANE), jnp.float32),
        jax.ShapeDtypeStruct((NWORK, NLANE), jnp.int32),
    ),
    scratch_types=[
        pltpu.VMEM((NZ,), jnp.float32),
        pltpu.VMEM((2, CH * NZ), jnp.float32),
        pltpu.VMEM((CH * NLANE,), jnp.float32),
        pltpu.VMEM((NLANE,), jnp.float32),
        pltpu.VMEM((NLANE,), jnp.int32),
        pltpu.SemaphoreType.DMA,
        pltpu.SemaphoreType.DMA,
    ],
)
def _sc_scan(x_hbm, w_hbm, val_hbm, idx_hbm, xb, wb, accb, vout, iout,
             sem0, sem1):
    cid = lax.axis_index("c")
    sid = lax.axis_index("s")
    wid = sid * 2 + cid
    base = wid * NPW
    sems = (sem0, sem1)

    pltpu.sync_copy(x_hbm, xb)
    xs = [xb[pl.ds(zc * NLANE, NLANE)] for zc in range(NZ // NLANE)]

    ivec = lax.iota(jnp.int32, NLANE)
    ivec16 = ivec * NLANE

    def chunk_src(ci):
        return w_hbm.at[pl.ds((base + ci * CH) * NZ, CH * NZ)]

    pltpu.async_copy(chunk_src(0), wb.at[0], sem0)

    def cbody(kk, carry):
        best, bidx = carry
        for b in (0, 1):
            ci = kk * 2 + b
            pltpu.make_async_copy(chunk_src(ci), wb.at[b], sems[b]).wait()

            @pl.when(ci + 1 < NCHUNK)
            def _():
                pltpu.async_copy(chunk_src(ci + 1), wb.at[1 - b], sems[1 - b])

            def gbody(g, _, b=b):
                goff = g * (NU * NZ)
                accs = []
                for k in range(NU):
                    acc = None
                    noff = goff + k * NZ
                    for zc in range(NZ // NLANE):
                        wv = wb[b, pl.ds(noff + zc * NLANE, NLANE)]
                        d = wv - xs[zc]
                        m = d * d
                        acc = m if acc is None else acc + m
                    accs.append(acc)
                aoff = g * (NU * NLANE)
                for k in range(NU):
                    accb[pl.ds(aoff + k * NLANE, NLANE)] = accs[k]
                return 0

            lax.fori_loop(0, NGRP, gbody, 0)

            row0 = base + ci * CH
            for g in range(CH // NLANE):
                tot = None
                for c in range(NLANE):
                    col = plsc.load_gather(
                        accb, [ivec16 + (g * NLANE * NLANE + c)])
                    tot = col if tot is None else tot + col
                gid = ivec + (row0 + g * NLANE)
                m = tot < best
                best = jnp.where(m, tot, best)
                bidx = jnp.where(m, gid, bidx)
        return best, bidx

    best = jnp.full((NLANE,), jnp.inf, jnp.float32)
    bidx = jnp.zeros((NLANE,), jnp.int32)
    best, bidx = lax.fori_loop(0, NCHUNK // 2, cbody, (best, bidx))

    vout[...] = best
    iout[...] = bidx
    pltpu.sync_copy(vout, val_hbm.at[wid])
    pltpu.sync_copy(iout, idx_hbm.at[wid])


def _tc_scan_body(x_ref, w_ref, vout_ref, iout_ref):
    g = pl.program_id(0)
    w = w_ref[...]
    xv = x_ref[...]
    d = w - xv[0]
    r2 = jnp.sum(d * d, axis=2)
    minv = jnp.min(r2)
    a = lax.broadcasted_iota(jnp.int32, (BTC // 128, 128), 0)
    b = lax.broadcasted_iota(jnp.int32, (BTC // 128, 128), 1)
    gbase = S_SC + g * BTC
    ids = a * 128 + b + gbase
    cand = jnp.where(r2 == minv, ids, jnp.int32(NN))
    flat = jnp.min(cand)
    vout_ref[...] = jnp.full((1, 1, 128), minv, jnp.float32)
    iout_ref[...] = jnp.full((1, 1, 128), flat, jnp.int32)


_tc_scan = pl.pallas_call(
    _tc_scan_body,
    grid=(GTC,),
    in_specs=[
        pl.BlockSpec((1, NZ), lambda g: (0, 0)),
        pl.BlockSpec((BTC // 128, 128, NZ), lambda g: (S_SC // BTC + g, 0, 0)),
    ],
    out_specs=[
        pl.BlockSpec((1, 1, 128), lambda g: (g, 0, 0)),
        pl.BlockSpec((1, 1, 128), lambda g: (g, 0, 0)),
    ],
    out_shape=[
        jax.ShapeDtypeStruct((GTC, 1, 128), jnp.float32),
        jax.ShapeDtypeStruct((GTC, 1, 128), jnp.int32),
    ],
)


def _tc_finish_body(sv_ref, si_ref, tv_ref, ti_ref, c_ref, o_ref):
    sv = sv_ref[...]
    si = si_ref[...]
    tv = tv_ref[...]
    ti = ti_ref[...]
    minv = jnp.minimum(jnp.min(sv), jnp.min(tv))
    big = jnp.int32(NN)
    flat = jnp.minimum(
        jnp.min(jnp.where(sv == minv, si, big)),
        jnp.min(jnp.where(tv == minv, ti, big)),
    )
    pf = (flat // NY).astype(jnp.float32)
    qf = (flat % NY).astype(jnp.float32)
    rows = lax.broadcasted_iota(jnp.int32, (NX, NY), 0).astype(jnp.float32)
    cols = lax.broadcasted_iota(jnp.int32, (NX, NY), 1).astype(jnp.float32)
    nv = c_ref[0]
    o_ref[...] = jnp.exp(((rows - pf) ** 2 + (cols - qf) ** 2) * nv)


_tc_finish = pl.pallas_call(
    _tc_finish_body,
    out_shape=jax.ShapeDtypeStruct((NX, NY), jnp.float32),
    in_specs=[
        pl.BlockSpec(memory_space=pltpu.VMEM),
        pl.BlockSpec(memory_space=pltpu.VMEM),
        pl.BlockSpec(memory_space=pltpu.VMEM),
        pl.BlockSpec(memory_space=pltpu.VMEM),
        pl.BlockSpec(memory_space=pltpu.SMEM),
    ],
    out_specs=pl.BlockSpec(memory_space=pltpu.VMEM),
)


def kernel(x, W, t):
    wf = W.reshape(NN * NZ)
    w3 = W.reshape(NN // 128, 128, NZ)
    x2 = x.reshape(1, NZ)
    tf = jnp.asarray(t).astype(jnp.float32)
    dec = SIG0 / (1.0 + tf / HALF_EPOCHS)
    ninv = (-1.0 / (2.0 * dec * dec)).reshape(1).astype(jnp.float32)
    sc_vals, sc_idx = _sc_scan(x, wf)
    tc_vals, tc_idx = _tc_scan(x2, w3)
    return _tc_finish(sc_vals, sc_idx, tc_vals, tc_idx, ninv)


# speedup vs baseline: 4.4410x; 1.9205x over previous
import functools

import jax
import jax.numpy as jnp
from jax import lax
from jax.experimental import pallas as pl
from jax.experimental.pallas import tpu as pltpu
from jax.experimental.pallas import tpu_sc as plsc

NZ = 256
NX = 256
NY = 256
NN = NX * NY
NWORK = 32
S_SC = 16384
NPW = S_SC // NWORK
CH = 128
NCHUNK = NPW // CH
NLANE = 16
NU = 8
NGRP = CH // NU
BTC = 2048
GTC = (NN - S_SC) // BTC
SIG0 = 0.8
HALF_EPOCHS = 50.0

_mesh = plsc.VectorSubcoreMesh(core_axis_name="c", subcore_axis_name="s")


@functools.partial(
    pl.kernel,
    mesh=_mesh,
    compiler_params=pltpu.CompilerParams(needs_layout_passes=False),
    out_type=(
        jax.ShapeDtypeStruct((NWORK, NLANE), jnp.float32),
        jax.ShapeDtypeStruct((NWORK, NLANE), jnp.int32),
    ),
    scratch_types=[
        pltpu.VMEM((NZ,), jnp.float32),
        pltpu.VMEM((2 * CH, NZ), jnp.float32),
        pltpu.VMEM((CH * NLANE,), jnp.float32),
        pltpu.VMEM((NLANE,), jnp.float32),
        pltpu.VMEM((NLANE,), jnp.int32),
        pltpu.SemaphoreType.DMA,
        pltpu.SemaphoreType.DMA,
    ],
)
def _sc_scan(x_hbm, w_hbm, val_hbm, idx_hbm, xb, wb, accb, vout, iout,
             sem0, sem1):
    cid = lax.axis_index("c")
    sid = lax.axis_index("s")
    wid = sid * 2 + cid
    base = wid * NPW
    sems = (sem0, sem1)

    pltpu.sync_copy(x_hbm, xb)
    xs = [xb[pl.ds(zc * NLANE, NLANE)] for zc in range(NZ // NLANE)]

    ivec = lax.iota(jnp.int32, NLANE)
    ivec16 = ivec * NLANE

    def chunk_src(ci):
        return w_hbm.at[pl.ds(base + ci * CH, CH)]

    def chunk_dst(b):
        return wb.at[pl.ds(b * CH, CH)]

    pltpu.async_copy(chunk_src(0), chunk_dst(0), sem0)

    def cbody(kk, carry):
        best, bidx = carry
        for b in (0, 1):
            ci = kk * 2 + b
            pltpu.make_async_copy(chunk_src(ci), chunk_dst(b), sems[b]).wait()

            @pl.when(ci + 1 < NCHUNK)
            def _():
                pltpu.async_copy(
                    chunk_src(ci + 1), chunk_dst(1 - b), sems[1 - b])

            def gbody(g, _, b=b):
                grow = b * CH + g * NU
                accs = []
                for k in range(NU):
                    acc = None
                    nrow = grow + k
                    for zc in range(NZ // NLANE):
                        wv = wb[nrow, pl.ds(zc * NLANE, NLANE)]
                        d = wv - xs[zc]
                        m = d * d
                        acc = m if acc is None else acc + m
                    accs.append(acc)
                aoff = g * (NU * NLANE)
                for k in range(NU):
                    accb[pl.ds(aoff + k * NLANE, NLANE)] = accs[k]
                return 0

            lax.fori_loop(0, NGRP, gbody, 0)

            row0 = base + ci * CH
            for g in range(CH // NLANE):
                tot = None
                for c in range(NLANE):
                    col = plsc.load_gather(
                        accb, [ivec16 + (g * NLANE * NLANE + c)])
                    tot = col if tot is None else tot + col
                gid = ivec + (row0 + g * NLANE)
                m = tot < best
                best = jnp.where(m, tot, best)
                bidx = jnp.where(m, gid, bidx)
        return best, bidx

    best = jnp.full((NLANE,), jnp.inf, jnp.float32)
    bidx = jnp.zeros((NLANE,), jnp.int32)
    best, bidx = lax.fori_loop(0, NCHUNK // 2, cbody, (best, bidx))

    vout[...] = best
    iout[...] = bidx
    pltpu.sync_copy(vout, val_hbm.at[wid])
    pltpu.sync_copy(iout, idx_hbm.at[wid])


def _tc_scan_body(x_ref, w_ref, vout_ref, iout_ref):
    g = pl.program_id(0)
    w = w_ref[...]
    xv = x_ref[...]
    d = w - xv[0]
    r2 = jnp.sum(d * d, axis=2)
    minv = jnp.min(r2)
    a = lax.broadcasted_iota(jnp.int32, (BTC // 128, 128), 0)
    b = lax.broadcasted_iota(jnp.int32, (BTC // 128, 128), 1)
    gbase = S_SC + g * BTC
    ids = a * 128 + b + gbase
    cand = jnp.where(r2 == minv, ids, jnp.int32(NN))
    flat = jnp.min(cand)
    vout_ref[...] = jnp.full((1, 1, 128), minv, jnp.float32)
    iout_ref[...] = jnp.full((1, 1, 128), flat, jnp.int32)


_tc_scan = pl.pallas_call(
    _tc_scan_body,
    grid=(GTC,),
    in_specs=[
        pl.BlockSpec((1, NZ), lambda g: (0, 0)),
        pl.BlockSpec((BTC // 128, 128, NZ), lambda g: (S_SC // BTC + g, 0, 0)),
    ],
    out_specs=[
        pl.BlockSpec((1, 1, 128), lambda g: (g, 0, 0)),
        pl.BlockSpec((1, 1, 128), lambda g: (g, 0, 0)),
    ],
    out_shape=[
        jax.ShapeDtypeStruct((GTC, 1, 128), jnp.float32),
        jax.ShapeDtypeStruct((GTC, 1, 128), jnp.int32),
    ],
)


def _tc_finish_body(sv_ref, si_ref, tv_ref, ti_ref, c_ref, o_ref):
    sv = sv_ref[...]
    si = si_ref[...]
    tv = tv_ref[...]
    ti = ti_ref[...]
    minv = jnp.minimum(jnp.min(sv), jnp.min(tv))
    big = jnp.int32(NN)
    flat = jnp.minimum(
        jnp.min(jnp.where(sv == minv, si, big)),
        jnp.min(jnp.where(tv == minv, ti, big)),
    )
    pf = (flat // NY).astype(jnp.float32)
    qf = (flat % NY).astype(jnp.float32)
    rows = lax.broadcasted_iota(jnp.int32, (NX, NY), 0).astype(jnp.float32)
    cols = lax.broadcasted_iota(jnp.int32, (NX, NY), 1).astype(jnp.float32)
    nv = c_ref[0]
    o_ref[...] = jnp.exp(((rows - pf) ** 2 + (cols - qf) ** 2) * nv)


_tc_finish = pl.pallas_call(
    _tc_finish_body,
    out_shape=jax.ShapeDtypeStruct((NX, NY), jnp.float32),
    in_specs=[
        pl.BlockSpec(memory_space=pltpu.VMEM),
        pl.BlockSpec(memory_space=pltpu.VMEM),
        pl.BlockSpec(memory_space=pltpu.VMEM),
        pl.BlockSpec(memory_space=pltpu.VMEM),
        pl.BlockSpec(memory_space=pltpu.SMEM),
    ],
    out_specs=pl.BlockSpec(memory_space=pltpu.VMEM),
)


def kernel(x, W, t):
    wf = W.reshape(NN, NZ)
    w3 = W.reshape(NN // 128, 128, NZ)
    x2 = x.reshape(1, NZ)
    tf = jnp.asarray(t).astype(jnp.float32)
    dec = SIG0 / (1.0 + tf / HALF_EPOCHS)
    ninv = (-1.0 / (2.0 * dec * dec)).reshape(1).astype(jnp.float32)
    sc_vals, sc_idx = _sc_scan(x, wf)
    tc_vals, tc_idx = _tc_scan(x2, w3)
    return _tc_finish(sc_vals, sc_idx, tc_vals, tc_idx, ninv)


# speedup vs baseline: 4.6826x; 1.0544x over previous
import functools

import jax
import jax.numpy as jnp
from jax import lax
from jax.experimental import pallas as pl
from jax.experimental.pallas import tpu as pltpu
from jax.experimental.pallas import tpu_sc as plsc

NZ = 256
NX = 256
NY = 256
NN = NX * NY
NWORK = 32
S_SC = 16384
NPW = S_SC // NWORK
CH = 128
NCHUNK = NPW // CH
NLANE = 16
NU = 8
NGRP = CH // NU
BTC = 2048
GTC = (NN - S_SC) // BTC
SIG0 = 0.8
HALF_EPOCHS = 50.0

_mesh = plsc.VectorSubcoreMesh(core_axis_name="c", subcore_axis_name="s")


@functools.partial(
    pl.kernel,
    mesh=_mesh,
    compiler_params=pltpu.CompilerParams(needs_layout_passes=False),
    out_type=(
        jax.ShapeDtypeStruct((NWORK, NLANE), jnp.float32),
        jax.ShapeDtypeStruct((NWORK, NLANE), jnp.int32),
    ),
    scratch_types=[
        pltpu.VMEM((NZ,), jnp.float32),
        pltpu.VMEM((2 * CH, NZ), jnp.float32),
        pltpu.VMEM((CH * NLANE,), jnp.float32),
        pltpu.VMEM((NLANE,), jnp.float32),
        pltpu.VMEM((NLANE,), jnp.int32),
        pltpu.SemaphoreType.DMA,
        pltpu.SemaphoreType.DMA,
    ],
)
def _sc_scan(x_hbm, w_hbm, val_hbm, idx_hbm, xb, wb, accb, vout, iout,
             sem0, sem1):
    cid = lax.axis_index("c")
    sid = lax.axis_index("s")
    wid = sid * 2 + cid
    base = wid * NPW
    sems = (sem0, sem1)

    pltpu.sync_copy(x_hbm, xb)
    xs = [xb[pl.ds(zc * NLANE, NLANE)] for zc in range(NZ // NLANE)]

    ivec = lax.iota(jnp.int32, NLANE)
    ivec16 = ivec * NLANE

    def chunk_src(ci):
        return w_hbm.at[pl.ds(base + ci * CH, CH)]

    def chunk_dst(b):
        return wb.at[pl.ds(b * CH, CH)]

    pltpu.async_copy(chunk_src(0), chunk_dst(0), sem0)

    def cbody(kk, carry):
        best, bidx = carry
        for b in (0, 1):
            ci = kk * 2 + b
            pltpu.make_async_copy(chunk_src(ci), chunk_dst(b), sems[b]).wait()

            @pl.when(ci + 1 < NCHUNK)
            def _():
                pltpu.async_copy(
                    chunk_src(ci + 1), chunk_dst(1 - b), sems[1 - b])

            def gbody(g, _, b=b):
                grow = b * CH + g * NU
                accs = []
                for k in range(NU):
                    acc = None
                    nrow = grow + k
                    for zc in range(NZ // NLANE):
                        wv = wb[nrow, pl.ds(zc * NLANE, NLANE)]
                        d = wv - xs[zc]
                        m = d * d
                        acc = m if acc is None else acc + m
                    accs.append(acc)
                aoff = g * (NU * NLANE)
                for k in range(NU):
                    accb[pl.ds(aoff + k * NLANE, NLANE)] = accs[k]
                return 0

            lax.fori_loop(0, NGRP, gbody, 0)

            row0 = base + ci * CH

            def tbody(g, bb):
                tbest, tbidx = bb
                colbase = g * (NLANE * NLANE)
                tot = None
                for c in range(NLANE):
                    col = plsc.load_gather(accb, [ivec16 + (colbase + c)])
                    tot = col if tot is None else tot + col
                gid = ivec + (row0 + g * NLANE)
                m = tot < tbest
                return jnp.where(m, tot, tbest), jnp.where(m, gid, tbidx)

            best, bidx = lax.fori_loop(0, CH // NLANE, tbody, (best, bidx))
        return best, bidx

    best = jnp.full((NLANE,), jnp.inf, jnp.float32)
    bidx = jnp.zeros((NLANE,), jnp.int32)
    best, bidx = lax.fori_loop(0, NCHUNK // 2, cbody, (best, bidx))

    vout[...] = best
    iout[...] = bidx
    pltpu.sync_copy(vout, val_hbm.at[wid])
    pltpu.sync_copy(iout, idx_hbm.at[wid])


def _tc_scan_body(x_ref, ones_ref, w_ref, vout_ref, iout_ref):
    g = pl.program_id(0)
    w = w_ref[...]
    xv = x_ref[...]
    d = w - xv[0]
    sq = d * d
    sp = sq[:, :, :128] + sq[:, :, 128:]
    sp2 = sp.reshape(BTC, 128)
    r2 = jax.lax.dot_general(
        sp2, ones_ref[...], (((1,), (0,)), ((), ())),
        preferred_element_type=jnp.float32)
    minv = jnp.min(r2)
    ids = lax.broadcasted_iota(jnp.int32, (BTC, 128), 0) + (S_SC + g * BTC)
    cand = jnp.where(r2 == minv, ids, jnp.int32(NN))
    flat = jnp.min(cand)
    vout_ref[...] = jnp.full((1, 1, 128), minv, jnp.float32)
    iout_ref[...] = jnp.full((1, 1, 128), flat, jnp.int32)


_tc_scan = pl.pallas_call(
    _tc_scan_body,
    grid=(GTC,),
    in_specs=[
        pl.BlockSpec((1, NZ), lambda g: (0, 0)),
        pl.BlockSpec((128, 128), lambda g: (0, 0)),
        pl.BlockSpec((BTC // 128, 128, NZ), lambda g: (S_SC // BTC + g, 0, 0)),
    ],
    out_specs=[
        pl.BlockSpec((1, 1, 128), lambda g: (g, 0, 0)),
        pl.BlockSpec((1, 1, 128), lambda g: (g, 0, 0)),
    ],
    out_shape=[
        jax.ShapeDtypeStruct((GTC, 1, 128), jnp.float32),
        jax.ShapeDtypeStruct((GTC, 1, 128), jnp.int32),
    ],
)


def _tc_finish_body(sv_ref, si_ref, tv_ref, ti_ref, c_ref, o_ref):
    sv = sv_ref[...]
    si = si_ref[...]
    tv = tv_ref[...]
    ti = ti_ref[...]
    minv = jnp.minimum(jnp.min(sv), jnp.min(tv))
    big = jnp.int32(NN)
    flat = jnp.minimum(
        jnp.min(jnp.where(sv == minv, si, big)),
        jnp.min(jnp.where(tv == minv, ti, big)),
    )
    pf = (flat // NY).astype(jnp.float32)
    qf = (flat % NY).astype(jnp.float32)
    rows = lax.broadcasted_iota(jnp.int32, (NX, NY), 0).astype(jnp.float32)
    cols = lax.broadcasted_iota(jnp.int32, (NX, NY), 1).astype(jnp.float32)
    nv = c_ref[0]
    o_ref[...] = jnp.exp(((rows - pf) ** 2 + (cols - qf) ** 2) * nv)


_tc_finish = pl.pallas_call(
    _tc_finish_body,
    out_shape=jax.ShapeDtypeStruct((NX, NY), jnp.float32),
    in_specs=[
        pl.BlockSpec(memory_space=pltpu.VMEM),
        pl.BlockSpec(memory_space=pltpu.VMEM),
        pl.BlockSpec(memory_space=pltpu.VMEM),
        pl.BlockSpec(memory_space=pltpu.VMEM),
        pl.BlockSpec(memory_space=pltpu.SMEM),
    ],
    out_specs=pl.BlockSpec(memory_space=pltpu.VMEM),
)


def kernel(x, W, t):
    wf = W.reshape(NN, NZ)
    w3 = W.reshape(NN // 128, 128, NZ)
    x2 = x.reshape(1, NZ)
    tf = jnp.asarray(t).astype(jnp.float32)
    dec = SIG0 / (1.0 + tf / HALF_EPOCHS)
    ninv = (-1.0 / (2.0 * dec * dec)).reshape(1).astype(jnp.float32)
    sc_vals, sc_idx = _sc_scan(x, wf)
    ones128 = jnp.ones((128, 128), jnp.float32)
    tc_vals, tc_idx = _tc_scan(x2, ones128, w3)
    return _tc_finish(sc_vals, sc_idx, tc_vals, tc_idx, ninv)
